# baseline (speedup 1.0000x reference)
import functools

import jax
import jax.numpy as jnp
from jax import lax
from jax.experimental import pallas as pl
from jax.experimental.pallas import tpu as pltpu
from jax.experimental.pallas import tpu_sc as plsc

HALF = 128
B, S = 4096, 50
N = B * S
NC, NS = 2, 16
NW = NC * NS
BPW = B // NW
ROWS_PER_W = BPW * S
XOFF = S * BPW
NBUF = 3
PAIRS = 2 * ROWS_PER_W


def _body(idx_hbm, x_hbm, y_hbm, out_hbm, idx_all, idx_u, obuf, gsem, wsem):
  wid = lax.axis_index("s") * NC + lax.axis_index("c")
  pltpu.sync_copy(idx_hbm.at[wid], idx_all)
  lanes = lax.iota(jnp.int32, 16)

  @pl.loop(0, S)
  def _(s):
    for p in range(BPW // 16):
      g = (p * 16 + lanes) * S + s
      idx_u[pl.ds(s * BPW + p * 16, 16)] = plsc.load_gather(idx_all, [2 * g])
      idx_u[pl.ds(XOFF + s * BPW + p * 16, 16)] = plsc.load_gather(
          idx_all, [2 * g + 1])

  def start_g(s, b):
    pltpu.async_copy(x_hbm.at[idx_u.at[pl.ds(s * BPW, BPW)]],
                     obuf.at[b, :, pl.ds(0, HALF)], gsem.at[b])
    pltpu.async_copy(y_hbm.at[idx_u.at[pl.ds(XOFF + s * BPW, BPW)]],
                     obuf.at[b, :, pl.ds(HALF, HALF)], gsem.at[b])

  def wait_g(b):
    pltpu.make_async_copy(x_hbm.at[idx_u.at[pl.ds(0, BPW)]],
                          obuf.at[b, :, pl.ds(0, HALF)], gsem.at[b]).wait()
    pltpu.make_async_copy(y_hbm.at[idx_u.at[pl.ds(0, BPW)]],
                          obuf.at[b, :, pl.ds(HALF, HALF)], gsem.at[b]).wait()

  def start_w(s, b):
    pltpu.async_copy(obuf.at[b], out_hbm.at[s, pl.ds(wid * BPW, BPW)],
                     wsem.at[b])

  def wait_w(b):
    pltpu.make_async_copy(obuf.at[b], out_hbm.at[0, pl.ds(0, BPW)],
                          wsem.at[b]).wait()

  def chunk(s, b):
    @pl.when(s >= 1)
    def _():
      wait_w((b + NBUF - 1) % NBUF)

    @pl.when(s + NBUF - 1 < S)
    def _():
      start_g(s + NBUF - 1, (b + NBUF - 1) % NBUF)

    wait_g(b)
    start_w(s, b)

  for c in range(NBUF - 1):
    start_g(c, c)

  @pl.loop(0, S - S % NBUF, step=NBUF)
  def _(s0):
    for b in range(NBUF):
      chunk(s0 + b, b)

  for s in range(S - S % NBUF, S):
    b = s % NBUF
    wait_w((b + NBUF - 1) % NBUF)
    if s + NBUF - 1 < S:
      start_g(s + NBUF - 1, (b + NBUF - 1) % NBUF)
    wait_g(b)
    start_w(s, b)

  wait_w((S - 1) % NBUF)


@functools.partial(jax.jit, donate_argnums=())
def _run(idx, x_table, y_table):
  mesh = plsc.VectorSubcoreMesh(core_axis_name="c", subcore_axis_name="s")
  kfn = pl.kernel(
      _body,
      out_type=jax.ShapeDtypeStruct((S, B, 2 * HALF), jnp.float32),
      mesh=mesh,
      scratch_types=[
          pltpu.VMEM((PAIRS,), jnp.int32),
          pltpu.VMEM((PAIRS,), jnp.int32),
          pltpu.VMEM((NBUF, BPW, 2 * HALF), jnp.float32),
          pltpu.SemaphoreType.DMA((NBUF,)),
          pltpu.SemaphoreType.DMA((NBUF,)),
      ],
      compiler_params=pltpu.CompilerParams(needs_layout_passes=False),
  )
  return kfn(idx, x_table, y_table)


def kernel(coords, x_table, y_table):
  idx = jnp.asarray(coords, jnp.int32).reshape(NW, PAIRS)
  out = _run(idx, x_table, y_table)
  return jnp.transpose(out, (1, 0, 2))


# speedup vs baseline: 1.0091x; 1.0091x over previous
import functools

import jax
import jax.numpy as jnp
from jax import lax
from jax.experimental import pallas as pl
from jax.experimental.pallas import tpu as pltpu
from jax.experimental.pallas import tpu_sc as plsc

HALF = 128
B, S = 4096, 50
N = B * S
NC, NS = 2, 16
NW = NC * NS
BPW = B // NW
ROWS_PER_W = BPW * S
XOFF = S * BPW
NBUF = 3
PAIRS = 2 * ROWS_PER_W


def _body(idx_hbm, x_hbm, y_hbm, out_hbm, idx_all, idx_u, obuf, gsem, wsem):
  wid = lax.axis_index("s") * NC + lax.axis_index("c")
  pltpu.sync_copy(idx_hbm.at[wid], idx_all)
  lanes = lax.iota(jnp.int32, 16)

  def unzip(s):
    for p in range(BPW // 16):
      g = (p * 16 + lanes) * S + s
      idx_u[pl.ds(s * BPW + p * 16, 16)] = plsc.load_gather(idx_all, [2 * g])
      idx_u[pl.ds(XOFF + s * BPW + p * 16, 16)] = plsc.load_gather(
          idx_all, [2 * g + 1])

  def start_g(s, b):
    pltpu.async_copy(x_hbm.at[idx_u.at[pl.ds(s * BPW, BPW)]],
                     obuf.at[b, :, pl.ds(0, HALF)], gsem.at[b])
    pltpu.async_copy(y_hbm.at[idx_u.at[pl.ds(XOFF + s * BPW, BPW)]],
                     obuf.at[b, :, pl.ds(HALF, HALF)], gsem.at[b])

  def wait_g(b):
    pltpu.make_async_copy(x_hbm.at[idx_u.at[pl.ds(0, BPW)]],
                          obuf.at[b, :, pl.ds(0, HALF)], gsem.at[b]).wait()
    pltpu.make_async_copy(y_hbm.at[idx_u.at[pl.ds(0, BPW)]],
                          obuf.at[b, :, pl.ds(HALF, HALF)], gsem.at[b]).wait()

  def start_w(s, b):
    pltpu.async_copy(obuf.at[b], out_hbm.at[s, pl.ds(wid * BPW, BPW)],
                     wsem.at[b])

  def wait_w(b):
    pltpu.make_async_copy(obuf.at[b], out_hbm.at[0, pl.ds(0, BPW)],
                          wsem.at[b]).wait()

  def chunk(s, b):
    @pl.when(s >= 1)
    def _():
      wait_w((b + NBUF - 1) % NBUF)

    @pl.when(s + NBUF - 1 < S)
    def _():
      unzip(s + NBUF - 1)
      start_g(s + NBUF - 1, (b + NBUF - 1) % NBUF)

    wait_g(b)
    start_w(s, b)

  for c in range(NBUF - 1):
    unzip(c)
    start_g(c, c)

  @pl.loop(0, S - S % NBUF, step=NBUF)
  def _(s0):
    for b in range(NBUF):
      chunk(s0 + b, b)

  for s in range(S - S % NBUF, S):
    b = s % NBUF
    wait_w((b + NBUF - 1) % NBUF)
    if s + NBUF - 1 < S:
      unzip(s + NBUF - 1)
      start_g(s + NBUF - 1, (b + NBUF - 1) % NBUF)
    wait_g(b)
    start_w(s, b)

  wait_w((S - 1) % NBUF)


@functools.partial(jax.jit, donate_argnums=())
def _run(idx, x_table, y_table):
  mesh = plsc.VectorSubcoreMesh(core_axis_name="c", subcore_axis_name="s")
  kfn = pl.kernel(
      _body,
      out_type=jax.ShapeDtypeStruct((S, B, 2 * HALF), jnp.float32),
      mesh=mesh,
      scratch_types=[
          pltpu.VMEM((PAIRS,), jnp.int32),
          pltpu.VMEM((PAIRS,), jnp.int32),
          pltpu.VMEM((NBUF, BPW, 2 * HALF), jnp.float32),
          pltpu.SemaphoreType.DMA((NBUF,)),
          pltpu.SemaphoreType.DMA((NBUF,)),
      ],
      compiler_params=pltpu.CompilerParams(needs_layout_passes=False),
  )
  return kfn(idx, x_table, y_table)


def kernel(coords, x_table, y_table):
  idx = jnp.asarray(coords, jnp.int32).reshape(NW, PAIRS)
  out = _run(idx, x_table, y_table)
  return jnp.transpose(out, (1, 0, 2))
